# Initial kernel scaffold; baseline (speedup 1.0000x reference)
import jax
import jax.numpy as jnp
from jax.experimental import pallas as pl


def kernel(x, edge_index, W1, a1_src, a1_dst, b1, W2, a2_src, a2_dst, b2):
    raise NotImplementedError("write your pallas kernel here")


# speedup vs baseline: 1.0519x; 1.0519x over previous
import functools

import jax
import jax.numpy as jnp
from jax.experimental import pallas as pl
from jax.experimental.pallas import tpu as pltpu

_N, _E, _DIN, _H, _C = 10000, 160000, 256, 4, 256


def _mm_alpha_body(x_ref, w_ref, aw_ref, h_ref, al_ref):
    h = jnp.dot(x_ref[...], w_ref[...], preferred_element_type=jnp.float32)
    h_ref[...] = h
    al_ref[...] = jnp.dot(h, aw_ref[...], preferred_element_type=jnp.float32)


def _matmul_alpha(x, W, A):
    N, K = x.shape
    BN = 400
    h, al = pl.pallas_call(
        _mm_alpha_body,
        grid=(N // BN,),
        in_specs=[
            pl.BlockSpec((BN, K), lambda i: (i, 0)),
            pl.BlockSpec((K, 1024), lambda i: (0, 0)),
            pl.BlockSpec((1024, 128), lambda i: (0, 0)),
        ],
        out_specs=[
            pl.BlockSpec((BN, 1024), lambda i: (i, 0)),
            pl.BlockSpec((BN, 128), lambda i: (i, 0)),
        ],
        out_shape=[
            jax.ShapeDtypeStruct((N, 1024), jnp.float32),
            jax.ShapeDtypeStruct((N, 128), jnp.float32),
        ],
    )(x, W, A)
    return h, al


def _edge_aggregate(h, alpha, src, dst):
    a_s = alpha[:, : _H]
    a_d = alpha[:, _H : 2 * _H]
    e = jax.nn.leaky_relu(a_s[src] + a_d[dst], negative_slope=0.2)
    w = jnp.exp(e)
    D = jax.ops.segment_sum(w, dst, num_segments=_N)
    hh = h.reshape(_N, _H, _C)
    msg = hh[src] * w[:, :, None]
    U = jax.ops.segment_sum(msg, dst, num_segments=_N).reshape(_N, _H * _C)
    return U, D


def kernel(x, edge_index, W1, a1_src, a1_dst, b1, W2, a2_src, a2_dst, b2):
    src = edge_index[0]
    dst = edge_index[1]

    def pack_alpha(a_src, a_dst):
        A = jnp.zeros((_H * _C, 128), jnp.float32)
        for hd in range(_H):
            A = A.at[hd * _C : (hd + 1) * _C, hd].set(a_src[hd])
            A = A.at[hd * _C : (hd + 1) * _C, _H + hd].set(a_dst[hd])
        return A

    A1 = pack_alpha(a1_src, a1_dst)
    A2 = pack_alpha(a2_src, a2_dst)

    h1, al1 = _matmul_alpha(x, W1, A1)
    U1, D1 = _edge_aggregate(h1, al1, src, dst)
    g1 = U1 / (jnp.repeat(D1, _C, axis=1) + 1e-16) + b1[None, :]
    g1 = jax.nn.elu(g1)

    h2, al2 = _matmul_alpha(g1, W2, A2)
    U2, D2 = _edge_aggregate(h2, al2, src, dst)
    g2 = U2.reshape(_N, _H, _C) / (D2[:, :, None] + 1e-16)
    g2 = g2.mean(axis=1) + b2[None, :]
    return jax.nn.elu(g2)


# speedup vs baseline: 11.7230x; 11.1441x over previous
import functools

import jax
import jax.numpy as jnp
from jax import lax
from jax.experimental import pallas as pl
from jax.experimental.pallas import tpu as pltpu
from jax.experimental.pallas import tpu_sc as plsc

_N, _E, _DIN, _H, _C = 10000, 160000, 256, 4, 256
_F = _H * _C

_HALF = 5000
_CH = 448
_NCH = 12
_DH = _CH * _NCH
_ACC_ROWS = 512
_TRASH = 511
_EPT = _E // 16
_NSEG = 5
_SEGE = _EPT // _NSEG
_SEGNB = _SEGE // 16
_CAP = _SEGE + 80


def _mm_alpha_body(x_ref, w_ref, aw_ref, h_ref, al_ref):
    h = jnp.dot(x_ref[...], w_ref[...], preferred_element_type=jnp.float32)
    h_ref[...] = h
    al_ref[...] = jnp.dot(h, aw_ref[...], preferred_element_type=jnp.float32)


def _matmul_alpha(x, W, A):
    N, K = x.shape
    BN = 400
    return pl.pallas_call(
        _mm_alpha_body,
        grid=(N // BN,),
        in_specs=[
            pl.BlockSpec((BN, K), lambda i: (i, 0)),
            pl.BlockSpec((K, _F), lambda i: (0, 0)),
            pl.BlockSpec((_F, 128), lambda i: (0, 0)),
        ],
        out_specs=[
            pl.BlockSpec((BN, _F), lambda i: (i, 0)),
            pl.BlockSpec((BN, 128), lambda i: (i, 0)),
        ],
        out_shape=[
            jax.ShapeDtypeStruct((N, _F), jnp.float32),
            jax.ShapeDtypeStruct((N, 128), jnp.float32),
        ],
    )(x, W, A)


_mesh = plsc.VectorSubcoreMesh(core_axis_name="c", subcore_axis_name="s")


@functools.partial(
    pl.kernel,
    out_type=[
        jax.ShapeDtypeStruct((2, _DH, _F), jnp.float32),
        jax.ShapeDtypeStruct((2, _DH, 16), jnp.float32),
    ],
    mesh=_mesh,
    scratch_types=[
        pltpu.VMEM((_H * _EPT,), jnp.float32),
        pltpu.VMEM((_CAP,), jnp.int32),
        pltpu.VMEM((_CAP,), jnp.int32),
        pltpu.VMEM((_CAP,), jnp.int32),
        pltpu.VMEM((_CAP,), jnp.int32),
        pltpu.VMEM((_CAP,), jnp.int32),
        pltpu.VMEM((16, _F), jnp.float32),
        pltpu.VMEM((16, 16), jnp.float32),
        pltpu.VMEM((4, _F), jnp.float32),
        pltpu.VMEM((4, 16), jnp.float32),
        pltpu.VMEM_SHARED((_ACC_ROWS, _F), jnp.float32),
        pltpu.VMEM_SHARED((_ACC_ROWS, 16), jnp.float32),
        pltpu.SemaphoreType.DMA,
    ],
    compiler_params=pltpu.CompilerParams(
        use_tc_tiling_on_sc=False, needs_layout_passes=False),
)
def _edge_call(h_hbm, alt_hbm, src_hbm, dst_hbm, u_hbm, d_hbm,
               w_v, sbuf, dbuf, csrc, cdl, ceid, G, wrow, zbuf, zbufd,
               acc, accD, gsem):
    sc = lax.axis_index("c")
    tid = lax.axis_index("s")
    half_lo = sc * _HALF
    ebase = tid * _EPT

    zero16f = jnp.zeros((16,), jnp.float32)
    zero16i = jnp.zeros((16,), jnp.int32)
    lane_i = lax.iota(jnp.int32, 16)

    def pass1(asv, adv):
        for hd in range(_H):
            pltpu.sync_copy(alt_hbm.at[hd], asv)
            pltpu.sync_copy(alt_hbm.at[_H + hd], adv)
            def p1seg(seg, c0, hd=hd):
                eoff = seg * _SEGE
                pltpu.sync_copy(
                    src_hbm.at[pl.ds(ebase + eoff, _SEGE)],
                    sbuf.at[pl.ds(0, _SEGE)])
                pltpu.sync_copy(
                    dst_hbm.at[pl.ds(ebase + eoff, _SEGE)],
                    dbuf.at[pl.ds(0, _SEGE)])

                def batch(b, c, hd=hd):
                    s16 = sbuf[pl.ds(b * 16, 16)]
                    d16 = dbuf[pl.ds(b * 16, 16)]
                    e = (plsc.load_gather(asv, [s16])
                         + plsc.load_gather(adv, [d16]))
                    e = jnp.where(e > 0, e, 0.2 * e)
                    w_v[pl.ds(hd * _EPT + eoff + b * 16, 16)] = jnp.exp(e)
                    return c
                lax.fori_loop(0, _SEGNB, batch, 0)
                return c0
            lax.fori_loop(0, _NSEG, p1seg, 0)

    pl.run_scoped(
        pass1,
        pltpu.VMEM((_N,), jnp.float32),
        pltpu.VMEM((_N,), jnp.float32),
    )

    for r in range(4):
        def zb(i, c, r=r):
            zbuf[r, pl.ds(i * 16, 16)] = zero16f
            return c
        lax.fori_loop(0, _F // 16, zb, 0)
        zbufd[r, pl.ds(0, 16)] = zero16f

    def chunk_body(ch, c00):
        cbase = half_lo + ch * _CH

        mbase = tid * (_ACC_ROWS // 16)
        def ms(i, c):
            pltpu.sync_copy(zbuf, acc.at[pl.ds(mbase + i * 4, 4)])
            pltpu.sync_copy(zbufd, accD.at[pl.ds(mbase + i * 4, 4)])
            return c
        lax.fori_loop(0, (_ACC_ROWS // 16) // 4, ms, 0)
        plsc.subcore_barrier()

        def seg_body(seg, c0):
            eoff = seg * _SEGE
            pltpu.sync_copy(
                src_hbm.at[pl.ds(ebase + eoff, _SEGE)],
                sbuf.at[pl.ds(0, _SEGE)])
            pltpu.sync_copy(
                dst_hbm.at[pl.ds(ebase + eoff, _SEGE)],
                dbuf.at[pl.ds(0, _SEGE)])

            def pf(i, c):
                csrc[pl.ds(i * 16, 16)] = zero16i
                cdl[pl.ds(i * 16, 16)] = jnp.full((16,), _TRASH, jnp.int32)
                ceid[pl.ds(i * 16, 16)] = zero16i
                return c
            lax.fori_loop(0, _CAP // 16, pf, 0)

            def compact(b, cnt):
                s16 = sbuf[pl.ds(b * 16, 16)]
                d16 = dbuf[pl.ds(b * 16, 16)]
                dl = d16 - cbase
                m = ((dl >= 0) & (dl < _CH)
                     & (d16 >= half_lo) & (d16 < half_lo + _HALF))
                pos = cnt + plsc.cumsum(m.astype(jnp.int32)) - 1
                pos = jnp.clip(pos, 0, _CAP - 1)
                plsc.store_scatter(csrc, [pos], s16, mask=m)
                plsc.store_scatter(cdl, [pos], jnp.clip(dl, 0, _CH - 1),
                                   mask=m)
                eid = lane_i + eoff + b * 16
                plsc.store_scatter(ceid, [pos], eid, mask=m)
                return cnt + plsc.all_reduce_population_count(m)

            cnt_sp = lax.fori_loop(0, _SEGNB, compact, zero16i)
            cnt = jnp.max(cnt_sp)
            nbat = (cnt + 15) // 16

            def proc(gi, c):
                sidx = csrc[pl.ds(gi * 16, 16)]
                pltpu.async_copy(h_hbm.at[sidx], G, gsem).wait()

                def pe(j, c2):
                    lane = jnp.full((16,), gi * 16 + j, jnp.int32)
                    eid16 = plsc.load_gather(ceid, [lane])
                    wv = zero16f
                    for hd in range(_H):
                        wsp = plsc.load_gather(w_v, [eid16 + hd * _EPT])
                        wv = jnp.where(lane_i == hd, wsp, wv)
                        for v in range(16):
                            col = hd * _C + v * 16
                            G[j, pl.ds(col, 16)] = G[j, pl.ds(col, 16)] * wsp
                    wrow[j, pl.ds(0, 16)] = wv
                    return c2
                lax.fori_loop(0, 16, pe, 0)

                didx = cdl[pl.ds(gi * 16, 16)]
                pltpu.sync_copy(G, acc.at[didx], add=True)
                pltpu.sync_copy(wrow, accD.at[didx], add=True)
                return c
            lax.fori_loop(0, nbat, proc, 0)
            return c0
        lax.fori_loop(0, _NSEG, seg_body, 0)

        plsc.subcore_barrier()
        rbase = tid * (_CH // 16)
        pltpu.sync_copy(
            acc.at[pl.ds(rbase, _CH // 16)],
            u_hbm.at[sc].at[pl.ds(ch * _CH + rbase, _CH // 16)])
        pltpu.sync_copy(
            accD.at[pl.ds(rbase, _CH // 16)],
            d_hbm.at[sc].at[pl.ds(ch * _CH + rbase, _CH // 16)])
        plsc.subcore_barrier()
        return c00
    lax.fori_loop(0, _NCH, chunk_body, 0)


def _norm1_body(u_ref, d_ref, b_ref, r4_ref, g_ref):
    drep = jnp.dot(d_ref[0], r4_ref[...], preferred_element_type=jnp.float32)
    g = u_ref[0] / (drep + 1e-16) + b_ref[0]
    g_ref[0] = jnp.where(g > 0, g, jnp.exp(g) - 1.0)


def _norm1(u, d, b, R4):
    BN = 448
    return pl.pallas_call(
        _norm1_body,
        grid=(2, _DH // BN),
        in_specs=[
            pl.BlockSpec((1, BN, _F), lambda s, i: (s, i, 0)),
            pl.BlockSpec((1, BN, 16), lambda s, i: (s, i, 0)),
            pl.BlockSpec((1, _F), lambda s, i: (0, 0)),
            pl.BlockSpec((16, _F), lambda s, i: (0, 0)),
        ],
        out_specs=pl.BlockSpec((1, BN, _F), lambda s, i: (s, i, 0)),
        out_shape=jax.ShapeDtypeStruct((2, _DH, _F), jnp.float32),
    )(u, d, b, R4)


def _norm2_body(u_ref, d_ref, b_ref, r4_ref, rm_ref, o_ref):
    drep = jnp.dot(d_ref[0], r4_ref[...], preferred_element_type=jnp.float32)
    g = u_ref[0] / (drep + 1e-16)
    gm = jnp.dot(g, rm_ref[...], preferred_element_type=jnp.float32) + b_ref[0]
    o_ref[0] = jnp.where(gm > 0, gm, jnp.exp(gm) - 1.0)


def _norm2(u, d, b, R4, Rm):
    BN = 448
    return pl.pallas_call(
        _norm2_body,
        grid=(2, _DH // BN),
        in_specs=[
            pl.BlockSpec((1, BN, _F), lambda s, i: (s, i, 0)),
            pl.BlockSpec((1, BN, 16), lambda s, i: (s, i, 0)),
            pl.BlockSpec((1, _C), lambda s, i: (0, 0)),
            pl.BlockSpec((16, _F), lambda s, i: (0, 0)),
            pl.BlockSpec((_F, _C), lambda s, i: (0, 0)),
        ],
        out_specs=pl.BlockSpec((1, BN, _C), lambda s, i: (s, i, 0)),
        out_shape=jax.ShapeDtypeStruct((2, _DH, _C), jnp.float32),
    )(u, d, b, R4, Rm)


def _pack_alpha(a_src, a_dst):
    A = jnp.zeros((_F, 128), jnp.float32)
    for hd in range(_H):
        A = A.at[hd * _C:(hd + 1) * _C, hd].set(a_src[hd])
        A = A.at[hd * _C:(hd + 1) * _C, _H + hd].set(a_dst[hd])
    return A


def kernel(x, edge_index, W1, a1_src, a1_dst, b1, W2, a2_src, a2_dst, b2):
    src = edge_index[0]
    dst = edge_index[1]

    A1 = _pack_alpha(a1_src, a1_dst)
    A2 = _pack_alpha(a2_src, a2_dst)
    R4 = jnp.zeros((16, _F), jnp.float32)
    for hd in range(_H):
        R4 = R4.at[hd, hd * _C:(hd + 1) * _C].set(1.0)
    Rm = jnp.tile(jnp.eye(_C, dtype=jnp.float32) / _H, (_H, 1))

    h1, al1 = _matmul_alpha(x, W1, A1)
    alt1 = jnp.transpose(al1[:, :2 * _H])
    u1, d1 = _edge_call(h1, alt1, src, dst)
    g1p = _norm1(u1, d1, b1.reshape(1, _F), R4)
    g1 = jnp.concatenate([g1p[0, :_HALF], g1p[1, :_HALF]], axis=0)

    h2, al2 = _matmul_alpha(g1, W2, A2)
    alt2 = jnp.transpose(al2[:, :2 * _H])
    u2, d2 = _edge_call(h2, alt2, src, dst)
    outp = _norm2(u2, d2, b2.reshape(1, _C), R4, Rm)
    return jnp.concatenate([outp[0, :_HALF], outp[1, :_HALF]], axis=0)
